# baseline (speedup 1.0000x reference)
import functools

import numpy as np
import jax
import jax.numpy as jnp
from jax import lax
from jax.experimental import pallas as pl
from jax.experimental.pallas import tpu as pltpu
from jax.experimental.pallas import tpu_sc as plsc

_MAX_LEN = 4096
_NC, _NS, _L = 2, 16, 16
_NW = _NC * _NS
_C = 16
_NBUF = 4


def _factor_tables(d_feature):
    h = d_feature // 2
    scale_factor = -np.log(10000.0) / (h - 1)
    div_term = np.exp(np.arange(0, h) * scale_factor)
    tab = np.empty((48, d_feature), dtype=np.float32)
    for block, mult in ((0, 256.0), (1, 16.0), (2, 1.0)):
        ang = (mult * np.arange(16))[:, None] * div_term[None, :]
        tab[16 * block:16 * (block + 1), :h] = np.sin(ang)
        tab[16 * block:16 * (block + 1), h:] = np.cos(ang)
    return jnp.asarray(tab)


def _sc_body(n_rows, d, x_hbm, pos_hbm, tab_hbm, out_hbm,
             tab_v, idx_raw, in0, in1, in2, in3,
             sem_in0, sem_in1, sem_in2, sem_in3,
             sem_out0, sem_out1, sem_out2, sem_out3):
    wid = lax.axis_index("s") * _NC + lax.axis_index("c")
    rows_per_w = n_rows // _NW
    base0 = wid * rows_per_w
    n_chunks = rows_per_w // _C
    n_wins = n_chunks // _NBUF
    h = d // 2
    n_grp = h // _L

    bufs = (in0, in1, in2, in3)
    sem_in = (sem_in0, sem_in1, sem_in2, sem_in3)
    sem_out = (sem_out0, sem_out1, sem_out2, sem_out3)

    pltpu.sync_copy(tab_hbm, tab_v)
    pltpu.sync_copy(pos_hbm.at[pl.ds(base0, rows_per_w)], idx_raw)

    def issue_in(c, j):
        return pltpu.async_copy(
            x_hbm.at[pl.ds(base0 + c * _C, _C)], bufs[j], sem_in[j])

    def wait_in(j):
        pltpu.make_async_copy(
            x_hbm.at[pl.ds(base0, _C)], bufs[j], sem_in[j]).wait()

    def wait_out(j):
        pltpu.make_async_copy(
            bufs[j], out_hbm.at[pl.ds(base0, _C)], sem_out[j]).wait()

    def compute(c, j):
        iv = bufs[j]
        idx_vec = idx_raw[pl.ds(pl.multiple_of(c * _C, _L), _L)]
        rs = []
        for t in range(_C):
            p = idx_vec[t]
            rs.append((lax.shift_right_logical(p, 8),
                       (lax.shift_right_logical(p, 4) & 15) + 16,
                       (p & 15) + 32))

        @plsc.parallel_loop(0, n_grp, 1, unroll=2)
        def _(k):
            off = pl.multiple_of(lax.shift_left(k, 4), _L)
            offh = pl.multiple_of(off + h, _L)
            for t in range(_C):
                rx, ry, rz = rs[t]
                sx = tab_v[rx, pl.ds(off, _L)]
                cx = tab_v[rx, pl.ds(offh, _L)]
                sy = tab_v[ry, pl.ds(off, _L)]
                cy = tab_v[ry, pl.ds(offh, _L)]
                sz = tab_v[rz, pl.ds(off, _L)]
                cz = tab_v[rz, pl.ds(offh, _L)]
                syz = sy * cz + cy * sz
                czy = cy * cz - sy * sz
                iv[t, pl.ds(off, _L)] = (
                    iv[t, pl.ds(off, _L)] + sx * czy + cx * syz)
                iv[t, pl.ds(offh, _L)] = (
                    iv[t, pl.ds(offh, _L)] + cx * czy - sx * syz)

        pltpu.async_copy(iv, out_hbm.at[pl.ds(base0 + c * _C, _C)],
                         sem_out[j])

    issue_in(0, 0)
    issue_in(1, 1)

    def window(i, carry):
        for j in range(_NBUF):
            c = i * _NBUF + j
            wait_in(j)
            compute(c, j)
            jn = (j + 2) % _NBUF
            if j < 2:
                @pl.when(i > 0)
                def _():
                    wait_out(jn)
                    issue_in(c + 2, jn)

                @pl.when(i == 0)
                def _():
                    issue_in(c + 2, jn)
            else:
                @pl.when(i < n_wins - 1)
                def _():
                    wait_out(jn)
                    issue_in(c + 2, jn)

                @pl.when(i == n_wins - 1)
                def _():
                    wait_out(jn)
        return carry

    lax.fori_loop(0, n_wins, window, 0)
    wait_out(2)
    wait_out(3)


def _make_sc_call(n_rows, d):
    mesh = plsc.VectorSubcoreMesh(
        core_axis_name="c", subcore_axis_name="s",
        num_cores=_NC, num_subcores=_NS)
    rows_per_w = n_rows // _NW
    return pl.kernel(
        functools.partial(_sc_body, n_rows, d),
        out_type=jax.ShapeDtypeStruct((n_rows, d), jnp.float32),
        mesh=mesh,
        scratch_types=[
            pltpu.VMEM((48, d), jnp.float32),
            pltpu.VMEM((rows_per_w,), jnp.int32),
            pltpu.VMEM((_C, d), jnp.float32),
            pltpu.VMEM((_C, d), jnp.float32),
            pltpu.VMEM((_C, d), jnp.float32),
            pltpu.VMEM((_C, d), jnp.float32),
            pltpu.SemaphoreType.DMA,
            pltpu.SemaphoreType.DMA,
            pltpu.SemaphoreType.DMA,
            pltpu.SemaphoreType.DMA,
            pltpu.SemaphoreType.DMA,
            pltpu.SemaphoreType.DMA,
            pltpu.SemaphoreType.DMA,
            pltpu.SemaphoreType.DMA,
        ],
    )


def kernel(inputs, inputs_positions):
    b, t, d = inputs.shape
    n_rows = b * t
    x = inputs.reshape(n_rows, d)
    pos = inputs_positions.reshape(n_rows).astype(jnp.int32)
    tab = _factor_tables(d)
    out = _make_sc_call(n_rows, d)(x, pos, tab)
    return out.reshape(b, t, d)


# speedup vs baseline: 1.2514x; 1.2514x over previous
import functools

import numpy as np
import jax
import jax.numpy as jnp
from jax import lax
from jax.experimental import pallas as pl
from jax.experimental.pallas import tpu as pltpu
from jax.experimental.pallas import tpu_sc as plsc

_MAX_LEN = 4096
_NC, _NS, _L = 2, 16, 16
_NW = _NC * _NS
_CT = 16
_CG = 8
_NWIN = 8


def _pe_table(d_feature):
    pe = np.zeros((_MAX_LEN, d_feature), dtype=np.float32)
    position = np.arange(0, _MAX_LEN)[:, np.newaxis]
    scale_factor = -np.log(10000.0) / (d_feature // 2 - 1)
    div_term = np.exp(np.arange(0, d_feature // 2) * scale_factor)
    pe[:, :d_feature // 2] = np.sin(position * div_term)
    pe[:, d_feature // 2:2 * (d_feature // 2)] = np.cos(position * div_term)
    return jnp.asarray(pe)


def _factor_tables(d_feature):
    h = d_feature // 2
    scale_factor = -np.log(10000.0) / (h - 1)
    div_term = np.exp(np.arange(0, h) * scale_factor)
    tab = np.empty((48, d_feature), dtype=np.float32)
    for block, mult in ((0, 256.0), (1, 16.0), (2, 1.0)):
        ang = (mult * np.arange(16))[:, None] * div_term[None, :]
        tab[16 * block:16 * (block + 1), :h] = np.sin(ang)
        tab[16 * block:16 * (block + 1), h:] = np.cos(ang)
    return jnp.asarray(tab)


def _sc_body(n_rows, d, x_hbm, pos_hbm, tab_hbm, pe_hbm, out_hbm,
             tab_v, idx_raw, t0, t1, g0, g1, p0, p1,
             sem_t0, sem_t1, sem_g0, sem_g1, sem_p0, sem_p1,
             sem_to0, sem_to1, sem_go0, sem_go1):
    wid = lax.axis_index("s") * _NC + lax.axis_index("c")
    rows_per_w = n_rows // _NW
    base0 = wid * rows_per_w
    gbase0 = base0 + _NWIN * 2 * _CT
    h = d // 2
    n_grp = h // _L

    tbufs = (t0, t1)
    gbufs = (g0, g1)
    pbufs = (p0, p1)
    sem_t = (sem_t0, sem_t1)
    sem_g = (sem_g0, sem_g1)
    sem_p = (sem_p0, sem_p1)
    sem_to = (sem_to0, sem_to1)
    sem_go = (sem_go0, sem_go1)

    pltpu.sync_copy(tab_hbm, tab_v)
    pltpu.sync_copy(pos_hbm.at[pl.ds(base0, rows_per_w)], idx_raw)

    def t_in(c, j):
        pltpu.async_copy(x_hbm.at[pl.ds(base0 + c * _CT, _CT)],
                         tbufs[j], sem_t[j])

    def t_in_wait(j):
        pltpu.make_async_copy(
            x_hbm.at[pl.ds(base0, _CT)], tbufs[j], sem_t[j]).wait()

    def t_out(c, j):
        pltpu.async_copy(tbufs[j], out_hbm.at[pl.ds(base0 + c * _CT, _CT)],
                         sem_to[j])

    def t_out_wait(j):
        pltpu.make_async_copy(
            tbufs[j], out_hbm.at[pl.ds(base0, _CT)], sem_to[j]).wait()

    def g_in(c, j):
        pltpu.async_copy(x_hbm.at[pl.ds(gbase0 + c * _CG, _CG)],
                         gbufs[j], sem_g[j])
        pltpu.async_copy(
            pe_hbm.at[idx_raw.at[pl.ds(_NWIN * 2 * _CT + c * _CG, _CG)]],
            pbufs[j], sem_p[j])

    def g_in_wait(j):
        pltpu.make_async_copy(
            x_hbm.at[pl.ds(gbase0, _CG)], gbufs[j], sem_g[j]).wait()
        pltpu.make_async_copy(
            pe_hbm.at[pl.ds(0, _CG)], pbufs[j], sem_p[j]).wait()

    def g_out(c, j):
        pltpu.async_copy(gbufs[j], out_hbm.at[pl.ds(gbase0 + c * _CG, _CG)],
                         sem_go[j])

    def g_out_wait(j):
        pltpu.make_async_copy(
            gbufs[j], out_hbm.at[pl.ds(gbase0, _CG)], sem_go[j]).wait()

    def table_compute(c, j):
        iv = tbufs[j]
        idx_vec = idx_raw[pl.ds(pl.multiple_of(c * _CT, _L), _L)]
        rs = []
        for t in range(_CT):
            p = idx_vec[t]
            rs.append((lax.shift_right_logical(p, 8),
                       (lax.shift_right_logical(p, 4) & 15) + 16,
                       (p & 15) + 32))

        @plsc.parallel_loop(0, n_grp, 1)
        def _(k):
            off = pl.multiple_of(lax.shift_left(k, 4), _L)
            offh = pl.multiple_of(off + h, _L)
            for t in range(_CT):
                rx, ry, rz = rs[t]
                sx = tab_v[rx, pl.ds(off, _L)]
                cx = tab_v[rx, pl.ds(offh, _L)]
                sy = tab_v[ry, pl.ds(off, _L)]
                cy = tab_v[ry, pl.ds(offh, _L)]
                sz = tab_v[rz, pl.ds(off, _L)]
                cz = tab_v[rz, pl.ds(offh, _L)]
                syz = sy * cz + cy * sz
                czy = cy * cz - sy * sz
                iv[t, pl.ds(off, _L)] = (
                    iv[t, pl.ds(off, _L)] + sx * czy + cx * syz)
                iv[t, pl.ds(offh, _L)] = (
                    iv[t, pl.ds(offh, _L)] + cx * czy - sx * syz)

    n_vec = d // _L
    shift_v = n_vec.bit_length() - 1
    assert (1 << shift_v) == n_vec

    def gather_add(j):
        iv, pv = gbufs[j], pbufs[j]

        @plsc.parallel_loop(0, _CG * n_vec, 1, unroll=8)
        def _(k):
            r = lax.shift_right_logical(k, shift_v)
            off = pl.multiple_of(lax.shift_left(k & (n_vec - 1), 4), _L)
            iv[r, pl.ds(off, _L)] = iv[r, pl.ds(off, _L)] + pv[r, pl.ds(off, _L)]

    def gather_part(c, i_is_first_chunk):
        b = c % 2
        nb = 1 - b
        if not i_is_first_chunk:
            g_out_wait(nb)

            @pl.when(c + 1 < 4 * _NWIN)
            def _():
                g_in(c + 1, nb)
        g_in_wait(b)
        gather_add(b)
        g_out(c, b)

    t_in(0, 0)
    t_in(1, 1)
    g_in(0, 0)
    g_in(1, 1)

    def window(i, carry):
        t_in_wait(0)
        table_compute(2 * i, 0)
        t_out(2 * i, 0)

        @pl.when(i > 0)
        def _():
            t_out_wait(1)
            t_in(2 * i + 1, 1)

        @pl.when(i > 0)
        def _():
            g_out_wait(0)
            g_in(4 * i, 0)
            g_out_wait(1)
            g_in(4 * i + 1, 1)

        g_in_wait(0)
        gather_add(0)
        g_out(4 * i, 0)

        g_out_wait(0)
        g_in(4 * i + 2, 0)
        g_in_wait(1)
        gather_add(1)
        g_out(4 * i + 1, 1)

        @pl.when(i < _NWIN - 1)
        def _():
            t_out_wait(0)
            t_in(2 * i + 2, 0)

        t_in_wait(1)
        table_compute(2 * i + 1, 1)
        t_out(2 * i + 1, 1)

        g_out_wait(1)
        g_in(4 * i + 3, 1)
        g_in_wait(0)
        gather_add(0)
        g_out(4 * i + 2, 0)

        g_in_wait(1)
        gather_add(1)
        g_out(4 * i + 3, 1)
        return carry

    lax.fori_loop(0, _NWIN, window, 0)
    t_out_wait(0)
    t_out_wait(1)
    g_out_wait(0)
    g_out_wait(1)


def _make_sc_call(n_rows, d):
    mesh = plsc.VectorSubcoreMesh(
        core_axis_name="c", subcore_axis_name="s",
        num_cores=_NC, num_subcores=_NS)
    rows_per_w = n_rows // _NW
    return pl.kernel(
        functools.partial(_sc_body, n_rows, d),
        out_type=jax.ShapeDtypeStruct((n_rows, d), jnp.float32),
        mesh=mesh,
        scratch_types=[
            pltpu.VMEM((48, d), jnp.float32),
            pltpu.VMEM((rows_per_w,), jnp.int32),
            pltpu.VMEM((_CT, d), jnp.float32),
            pltpu.VMEM((_CT, d), jnp.float32),
            pltpu.VMEM((_CG, d), jnp.float32),
            pltpu.VMEM((_CG, d), jnp.float32),
            pltpu.VMEM((_CG, d), jnp.float32),
            pltpu.VMEM((_CG, d), jnp.float32),
        ] + [pltpu.SemaphoreType.DMA] * 10,
    )


def kernel(inputs, inputs_positions):
    b, t, d = inputs.shape
    n_rows = b * t
    x = inputs.reshape(n_rows, d)
    pos = inputs_positions.reshape(n_rows).astype(jnp.int32)
    tab = _factor_tables(d)
    pe = _pe_table(d)
    out = _make_sc_call(n_rows, d)(x, pos, tab, pe)
    return out.reshape(b, t, d)


# speedup vs baseline: 1.6748x; 1.3384x over previous
import functools

import numpy as np
import jax
import jax.numpy as jnp
from jax import lax
from jax.experimental import pallas as pl
from jax.experimental.pallas import tpu as pltpu
from jax.experimental.pallas import tpu_sc as plsc

_MAX_LEN = 4096
_NC, _NS, _L = 2, 16, 16
_NW = _NC * _NS
_C = 16


def _pe_table(d_feature):
    pe = np.zeros((_MAX_LEN, d_feature), dtype=np.float32)
    position = np.arange(0, _MAX_LEN)[:, np.newaxis]
    scale_factor = -np.log(10000.0) / (d_feature // 2 - 1)
    div_term = np.exp(np.arange(0, d_feature // 2) * scale_factor)
    pe[:, :d_feature // 2] = np.sin(position * div_term)
    pe[:, d_feature // 2:2 * (d_feature // 2)] = np.cos(position * div_term)
    return jnp.asarray(pe)


def _sc_body(n_rows, d, x_hbm, pos_hbm, pe_hbm, out_hbm,
             idx_all, in0, in1, pe0, pe1,
             sem_in0, sem_in1, sem_pe0, sem_pe1, sem_out0, sem_out1):
    wid = lax.axis_index("s") * _NC + lax.axis_index("c")
    rows_per_w = n_rows // _NW
    base0 = wid * rows_per_w
    n_chunks = rows_per_w // _C
    n_vec = d // _L
    shift_r = n_vec.bit_length() - 1
    assert (1 << shift_r) == n_vec
    n_vec_chunk = _C * n_vec

    in_b = (in0, in1)
    pe_b = (pe0, pe1)
    sem_in = (sem_in0, sem_in1)
    sem_pe = (sem_pe0, sem_pe1)
    sem_out = (sem_out0, sem_out1)

    pltpu.sync_copy(pos_hbm.at[pl.ds(base0, rows_per_w)], idx_all)

    def issue(g):
        b = g & 1
        base = base0 + g * _C
        ci = pltpu.async_copy(x_hbm.at[pl.ds(base, _C)], in_b[b], sem_in[b])
        cp = pltpu.async_copy(pe_hbm.at[idx_all.at[pl.ds(g * _C, _C)]],
                              pe_b[b], sem_pe[b])
        return ci, cp

    pending = [None, None]
    out_desc = [None, None]
    pending[0] = issue(0)

    for g in range(n_chunks):
        b = g & 1
        ci, cp = pending[b]
        ci.wait()
        cp.wait()
        if g + 1 < n_chunks:
            if out_desc[1 - b] is not None:
                out_desc[1 - b].wait()
            pending[1 - b] = issue(g + 1)

        iv, pv = in_b[b], pe_b[b]

        @plsc.parallel_loop(0, n_vec_chunk, 1, unroll=8)
        def _(k):
            r = lax.shift_right_logical(k, shift_r)
            off = pl.multiple_of(lax.shift_left(k & (n_vec - 1), 4), _L)
            iv[r, pl.ds(off, _L)] = iv[r, pl.ds(off, _L)] + pv[r, pl.ds(off, _L)]

        out_desc[b] = pltpu.async_copy(
            iv, out_hbm.at[pl.ds(base0 + g * _C, _C)], sem_out[b])

    for b in range(2):
        if out_desc[b] is not None:
            out_desc[b].wait()


def _make_sc_call(n_rows, d):
    mesh = plsc.VectorSubcoreMesh(
        core_axis_name="c", subcore_axis_name="s",
        num_cores=_NC, num_subcores=_NS)
    return pl.kernel(
        functools.partial(_sc_body, n_rows, d),
        out_type=jax.ShapeDtypeStruct((n_rows, d), jnp.float32),
        mesh=mesh,
        scratch_types=[
            pltpu.VMEM((n_rows // _NW,), jnp.int32),
            pltpu.VMEM((_C, d), jnp.float32),
            pltpu.VMEM((_C, d), jnp.float32),
            pltpu.VMEM((_C, d), jnp.float32),
            pltpu.VMEM((_C, d), jnp.float32),
            pltpu.SemaphoreType.DMA,
            pltpu.SemaphoreType.DMA,
            pltpu.SemaphoreType.DMA,
            pltpu.SemaphoreType.DMA,
            pltpu.SemaphoreType.DMA,
            pltpu.SemaphoreType.DMA,
        ],
    )


def kernel(inputs, inputs_positions):
    b, t, d = inputs.shape
    n_rows = b * t
    x = inputs.reshape(n_rows, d)
    pos = inputs_positions.reshape(n_rows).astype(jnp.int32)
    pe = _pe_table(d)
    out = _make_sc_call(n_rows, d)(x, pos, pe)
    return out.reshape(b, t, d)


# speedup vs baseline: 1.7357x; 1.0363x over previous
import functools

import numpy as np
import jax
import jax.numpy as jnp
from jax import lax
from jax.experimental import pallas as pl
from jax.experimental.pallas import tpu as pltpu
from jax.experimental.pallas import tpu_sc as plsc

_MAX_LEN = 4096
_NC, _NS, _L = 2, 16, 16
_NW = _NC * _NS
_C = 16


def _pe_table(d_feature):
    pe = np.zeros((_MAX_LEN, d_feature), dtype=np.float32)
    position = np.arange(0, _MAX_LEN)[:, np.newaxis]
    scale_factor = -np.log(10000.0) / (d_feature // 2 - 1)
    div_term = np.exp(np.arange(0, d_feature // 2) * scale_factor)
    pe[:, :d_feature // 2] = np.sin(position * div_term)
    pe[:, d_feature // 2:2 * (d_feature // 2)] = np.cos(position * div_term)
    return jnp.asarray(pe)


def _sc_body(n_rows, d, x_hbm, pos_hbm, pe_hbm, out_hbm,
             idx_all, in0, in1, in2, pe0, pe1, pe2,
             sem_in0, sem_in1, sem_in2, sem_pe0, sem_pe1, sem_pe2,
             sem_out0, sem_out1, sem_out2):
    wid = lax.axis_index("s") * _NC + lax.axis_index("c")
    rows_per_w = n_rows // _NW
    base0 = wid * rows_per_w
    n_chunks = rows_per_w // _C
    n_vec = d // _L
    shift_r = n_vec.bit_length() - 1
    assert (1 << shift_r) == n_vec
    n_vec_chunk = _C * n_vec

    in_b = (in0, in1, in2)
    pe_b = (pe0, pe1, pe2)
    sem_in = (sem_in0, sem_in1, sem_in2)
    sem_pe = (sem_pe0, sem_pe1, sem_pe2)
    sem_out = (sem_out0, sem_out1, sem_out2)
    nbuf = 3

    pltpu.sync_copy(pos_hbm.at[pl.ds(base0, rows_per_w)], idx_all)

    def issue(g):
        b = g % nbuf
        base = base0 + g * _C
        ci = pltpu.async_copy(x_hbm.at[pl.ds(base, _C)], in_b[b], sem_in[b])
        cp = pltpu.async_copy(pe_hbm.at[idx_all.at[pl.ds(g * _C, _C)]],
                              pe_b[b], sem_pe[b])
        return ci, cp

    pending = [None] * nbuf
    out_desc = [None] * nbuf
    pending[0] = issue(0)
    pending[1] = issue(1)

    for g in range(n_chunks):
        b = g % nbuf
        ci, cp = pending[b]
        ci.wait()
        cp.wait()
        if g + 2 < n_chunks:
            nb = (g + 2) % nbuf
            if out_desc[nb] is not None:
                out_desc[nb].wait()
            pending[nb] = issue(g + 2)

        iv, pv = in_b[b], pe_b[b]

        @plsc.parallel_loop(0, n_vec_chunk, 1, unroll=8)
        def _(k):
            r = lax.shift_right_logical(k, shift_r)
            off = pl.multiple_of(lax.shift_left(k & (n_vec - 1), 4), _L)
            iv[r, pl.ds(off, _L)] = iv[r, pl.ds(off, _L)] + pv[r, pl.ds(off, _L)]

        out_desc[b] = pltpu.async_copy(
            iv, out_hbm.at[pl.ds(base0 + g * _C, _C)], sem_out[b])

    for b in range(nbuf):
        if out_desc[b] is not None:
            out_desc[b].wait()


def _make_sc_call(n_rows, d):
    mesh = plsc.VectorSubcoreMesh(
        core_axis_name="c", subcore_axis_name="s",
        num_cores=_NC, num_subcores=_NS)
    return pl.kernel(
        functools.partial(_sc_body, n_rows, d),
        out_type=jax.ShapeDtypeStruct((n_rows, d), jnp.float32),
        mesh=mesh,
        scratch_types=[
            pltpu.VMEM((n_rows // _NW,), jnp.int32),
        ] + [pltpu.VMEM((_C, d), jnp.float32)] * 6
          + [pltpu.SemaphoreType.DMA] * 9,
    )


def kernel(inputs, inputs_positions):
    b, t, d = inputs.shape
    n_rows = b * t
    x = inputs.reshape(n_rows, d)
    pos = inputs_positions.reshape(n_rows).astype(jnp.int32)
    pe = _pe_table(d)
    out = _make_sc_call(n_rows, d)(x, pos, pe)
    return out.reshape(b, t, d)
